# baseline (speedup 1.0000x reference)
import jax
import jax.numpy as jnp
from jax import lax
from jax.experimental import pallas as pl
from jax.experimental.pallas import tpu as pltpu
from jax.experimental.pallas import tpu_sc as plsc

N = 10000
E = 320000
D = 128

NC = 2
NS = 16
NW = NC * NS
L = 16

B = 40
EDGES_PER_W = E // NW
CPW = EDGES_PER_W // B
NPOS = 6
NPB = 41
TAIL0 = NPOS * NPB
ZROWS = 80
NBLK = N // ZROWS


def _sc_body(w_hbm, x_hbm, src_hbm, dst_hbm, parts_hbm, acc_sh,
             s0, s1, s2, s3, s4, s5, d0, d1, d2, d3, d4, d5,
             w0, w1, w2, x0, x1, x2, z0,
             is0, is1, is2, is3, is4, is5,
             ls0, ls1, ls2, gs0, gs1, gs2, ss0, ss1, ss2):
    cid = lax.axis_index("c")
    sid = lax.axis_index("s")
    wid = sid * NC + cid

    srcb = [s0, s1, s2, s3, s4, s5]
    dstb = [d0, d1, d2, d3, d4, d5]
    wbuf = [w0, w1, w2]
    xbuf = [x0, x1, x2]
    isem = [is0, is1, is2, is3, is4, is5]
    lsem = [ls0, ls1, ls2]
    gsem = [gs0, gs1, gs2]
    ssem = [ss0, ss1, ss2]

    @plsc.parallel_loop(0, ZROWS, unroll=8)
    def _zero_row(r):
        for j in range(D // L):
            z0[r, pl.ds(j * L, L)] = jnp.zeros((L,), jnp.float32)
    for k in range((NBLK + NS - 1) // NS):
        blk = sid + k * NS
        @pl.when(blk < NBLK)
        def _():
            pltpu.sync_copy(z0, acc_sh.at[pl.ds(blk * ZROWS, ZROWS)])
    plsc.subcore_barrier()

    e_base = wid * EDGES_PER_W

    def idx_start(i, slot):
        e0 = e_base + i * B
        pltpu.async_copy(src_hbm.at[pl.ds(e0, B)], srcb[slot], isem[slot])
        pltpu.async_copy(dst_hbm.at[pl.ds(e0, B)], dstb[slot], isem[slot])

    def idx_wait(slot):
        pltpu.make_async_copy(src_hbm.at[pl.ds(0, B)], srcb[slot],
                              isem[slot]).wait()
        pltpu.make_async_copy(dst_hbm.at[pl.ds(0, B)], dstb[slot],
                              isem[slot]).wait()

    def wload_start(i, buf):
        pltpu.async_copy(w_hbm.at[pl.ds(e_base + i * B, B)], wbuf[buf],
                         lsem[buf])

    def gather_start(slot, buf):
        pltpu.async_copy(x_hbm.at[srcb[slot]], xbuf[buf], gsem[buf])

    def data_wait(buf):
        pltpu.make_async_copy(w_hbm.at[pl.ds(0, B)], wbuf[buf],
                              lsem[buf]).wait()
        pltpu.make_async_copy(x_hbm.at[srcb[0]], xbuf[buf],
                              gsem[buf]).wait()

    def scat_start(slot, buf):
        pltpu.async_copy(xbuf[buf], acc_sh.at[dstb[slot]], ssem[buf],
                         add=True)

    def scat_wait(buf):
        pltpu.make_async_copy(xbuf[buf], acc_sh.at[dstb[0]],
                              ssem[buf]).wait()

    def compute(buf):
        wb = wbuf[buf]
        xb = xbuf[buf]
        @plsc.parallel_loop(0, B, unroll=8)
        def _row(r):
            for j in range(D // L):
                sl = pl.ds(j * L, L)
                xb[r, sl] = xb[r, sl] * wb[r, sl]

    def position(c, pos, q):
        slot = pos % NPOS
        nslot = (pos + 1) % NPOS
        pslot = (pos + 2) % NPOS
        buf = pos % 3
        nbuf = (pos + 1) % 3
        if q is None:
            if c + 2 < CPW:
                idx_start(c + 2, pslot)
            if c + 1 < CPW:
                wload_start(c + 1, nbuf)
            scat_wait(nbuf)
            if c + 1 < CPW:
                idx_wait(nslot)
                gather_start(nslot, nbuf)
            data_wait(buf)
            compute(buf)
            scat_start(slot, buf)
        else:
            idx_start(c + 2, pslot)
            wload_start(c + 1, nbuf)
            if pos <= 1:
                @pl.when(q > 0)
                def _():
                    scat_wait(nbuf)
            else:
                scat_wait(nbuf)
            idx_wait(nslot)
            gather_start(nslot, nbuf)
            data_wait(buf)
            compute(buf)
            scat_start(slot, buf)

    idx_start(0, 0)
    idx_start(1, 1)
    idx_wait(0)
    wload_start(0, 0)
    gather_start(0, 0)

    def _body(q, _):
        c0 = NPOS * q
        for pos in range(NPOS):
            position(c0 + pos, pos, q)
        return 0

    lax.fori_loop(0, NPB, _body, 0)

    for c in range(TAIL0, CPW):
        position(c, c % NPOS, None)
    scat_wait((CPW - 2) % 3)
    scat_wait((CPW - 1) % 3)
    plsc.subcore_barrier()

    for k in range((NBLK + NS - 1) // NS):
        blk = sid + k * NS
        @pl.when(blk < NBLK)
        def _():
            r0 = blk * ZROWS
            pltpu.sync_copy(acc_sh.at[pl.ds(r0, ZROWS)],
                            parts_hbm.at[cid, pl.ds(r0, ZROWS)])


@jax.jit
def _sc_scatter(w, x, src, dst):
    mesh = plsc.VectorSubcoreMesh(core_axis_name="c", subcore_axis_name="s")
    return pl.kernel(
        _sc_body,
        out_type=jax.ShapeDtypeStruct((NC, N, D), jnp.float32),
        mesh=mesh,
        scratch_types=(
            [pltpu.VMEM_SHARED((N, D), jnp.float32)]
            + [pltpu.VMEM((B,), jnp.int32) for _ in range(12)]
            + [pltpu.VMEM((B, D), jnp.float32) for _ in range(6)]
            + [pltpu.VMEM((ZROWS, D), jnp.float32)]
            + [pltpu.SemaphoreType.DMA for _ in range(15)]
        ),
    )(w, x, src, dst)


def _add_body(a_ref, b_ref, o_ref):
    o_ref[...] = a_ref[0] + b_ref[0]


@jax.jit
def _combine(parts):
    blk = 1000
    return pl.pallas_call(
        _add_body,
        grid=(N // blk,),
        in_specs=[
            pl.BlockSpec((1, blk, D), lambda i: (0, i, 0)),
            pl.BlockSpec((1, blk, D), lambda i: (1, i, 0)),
        ],
        out_specs=pl.BlockSpec((blk, D), lambda i: (i, 0)),
        out_shape=jax.ShapeDtypeStruct((N, D), jnp.float32),
    )(parts, parts)


def kernel(w, x, src, dst):
    parts = _sc_scatter(w, x, src, dst)
    return _combine(parts)


# speedup vs baseline: 1.2001x; 1.2001x over previous
import jax
import jax.numpy as jnp
from jax import lax
from jax.experimental import pallas as pl
from jax.experimental.pallas import tpu as pltpu
from jax.experimental.pallas import tpu_sc as plsc

N = 10000
E = 320000
D = 128

NC = 2
NS = 16
NW = NC * NS
L = 16

B = 80
EDGES_PER_W = E // NW
CPW = EDGES_PER_W // B
NPB = (CPW - 1) // 4
ZROWS = B
NBLK = N // ZROWS


def _sc_body(w_hbm, x_hbm, src_hbm, dst_hbm, parts_hbm, acc_sh,
             s0, s1, s2, s3, d0, d1, d2, d3, w0, w1, x0, x1,
             is0, is1, is2, is3, ls0, ls1, gs0, gs1, ss0, ss1):
    cid = lax.axis_index("c")
    sid = lax.axis_index("s")
    wid = sid * NC + cid

    srcb = [s0, s1, s2, s3]
    dstb = [d0, d1, d2, d3]
    wbuf = [w0, w1]
    xbuf = [x0, x1]
    isem = [is0, is1, is2, is3]
    lsem = [ls0, ls1]
    gsem = [gs0, gs1]
    ssem = [ss0, ss1]

    @plsc.parallel_loop(0, ZROWS, unroll=4)
    def _zero_row(r):
        for j in range(D // L):
            w0[r, pl.ds(j * L, L)] = jnp.zeros((L,), jnp.float32)
    for k in range((NBLK + NS - 1) // NS):
        blk = sid + k * NS
        @pl.when(blk < NBLK)
        def _():
            pltpu.sync_copy(w0, acc_sh.at[pl.ds(blk * ZROWS, ZROWS)])
    plsc.subcore_barrier()

    e_base = wid * EDGES_PER_W

    def idx_start(i, slot):
        e0 = e_base + i * B
        pltpu.async_copy(src_hbm.at[pl.ds(e0, B)], srcb[slot], isem[slot])
        pltpu.async_copy(dst_hbm.at[pl.ds(e0, B)], dstb[slot], isem[slot])

    def idx_wait(slot):
        pltpu.make_async_copy(src_hbm.at[pl.ds(0, B)], srcb[slot],
                              isem[slot]).wait()
        pltpu.make_async_copy(dst_hbm.at[pl.ds(0, B)], dstb[slot],
                              isem[slot]).wait()

    def wload_start(i, buf):
        pltpu.async_copy(w_hbm.at[pl.ds(e_base + i * B, B)], wbuf[buf],
                         lsem[buf])

    def gather_start(slot, buf):
        pltpu.async_copy(x_hbm.at[srcb[slot]], xbuf[buf], gsem[buf])

    def data_wait(buf):
        pltpu.make_async_copy(w_hbm.at[pl.ds(0, B)], wbuf[buf],
                              lsem[buf]).wait()
        pltpu.make_async_copy(x_hbm.at[srcb[0]], xbuf[buf],
                              gsem[buf]).wait()

    def scat_start(i, slot, buf):
        pltpu.async_copy(wbuf[buf], acc_sh.at[dstb[slot]], ssem[buf],
                         add=True)

    def scat_wait(buf):
        pltpu.make_async_copy(wbuf[buf], acc_sh.at[dstb[0]],
                              ssem[buf]).wait()

    def compute(buf):
        wb = wbuf[buf]
        xb = xbuf[buf]
        @plsc.parallel_loop(0, B, unroll=8)
        def _row(r):
            for j in range(D // L):
                sl = pl.ds(j * L, L)
                wb[r, sl] = xb[r, sl] * wb[r, sl]

    idx_start(0, 0)
    idx_start(1, 1)
    idx_wait(0)
    wload_start(0, 0)
    gather_start(0, 0)

    def _body(q, _):
        c0 = 4 * q
        for pos in range(4):
            c = c0 + pos
            slot = pos % 4
            nslot = (pos + 1) % 4
            pslot = (pos + 2) % 4
            buf = pos % 2
            nbuf = (pos + 1) % 2
            if pos == 3:
                @pl.when(q < NPB - 1)
                def _():
                    idx_start(c + 2, pslot)
            else:
                idx_start(c + 2, pslot)
            idx_wait(nslot)
            gather_start(nslot, nbuf)
            if pos == 0:
                @pl.when(q > 0)
                def _():
                    scat_wait(nbuf)
            else:
                scat_wait(nbuf)
            wload_start(c + 1, nbuf)
            data_wait(buf)
            compute(buf)
            scat_start(c, slot, buf)
        return 0

    lax.fori_loop(0, NPB, _body, 0)

    scat_wait(1)
    data_wait(0)
    compute(0)
    scat_start(CPW - 1, 0, 0)
    scat_wait(0)
    plsc.subcore_barrier()

    for k in range((NBLK + NS - 1) // NS):
        blk = sid + k * NS
        @pl.when(blk < NBLK)
        def _():
            r0 = blk * ZROWS
            pltpu.sync_copy(acc_sh.at[pl.ds(r0, ZROWS)],
                            parts_hbm.at[cid, pl.ds(r0, ZROWS)])


@jax.jit
def _sc_scatter(w, x, src, dst):
    mesh = plsc.VectorSubcoreMesh(core_axis_name="c", subcore_axis_name="s")
    return pl.kernel(
        _sc_body,
        out_type=jax.ShapeDtypeStruct((NC, N, D), jnp.float32),
        mesh=mesh,
        scratch_types=(
            [pltpu.VMEM_SHARED((N, D), jnp.float32)]
            + [pltpu.VMEM((B,), jnp.int32) for _ in range(8)]
            + [pltpu.VMEM((B, D), jnp.float32) for _ in range(4)]
            + [pltpu.SemaphoreType.DMA for _ in range(10)]
        ),
    )(w, x, src, dst)


def _add_body(a_ref, b_ref, o_ref):
    o_ref[...] = a_ref[0] + b_ref[0]


@jax.jit
def _combine(parts):
    blk = 2000
    return pl.pallas_call(
        _add_body,
        grid=(N // blk,),
        in_specs=[
            pl.BlockSpec((1, blk, D), lambda i: (0, i, 0)),
            pl.BlockSpec((1, blk, D), lambda i: (1, i, 0)),
        ],
        out_specs=pl.BlockSpec((blk, D), lambda i: (i, 0)),
        out_shape=jax.ShapeDtypeStruct((N, D), jnp.float32),
    )(parts, parts)


def kernel(w, x, src, dst):
    parts = _sc_scatter(w, x, src, dst)
    return _combine(parts)
